# Initial kernel scaffold; baseline (speedup 1.0000x reference)
import jax
import jax.numpy as jnp
from jax.experimental import pallas as pl


def kernel(user_input, item_input, item_embedding, user_embedding):
    raise NotImplementedError("write your pallas kernel here")


# speedup vs baseline: 2.3162x; 2.3162x over previous
import functools

import jax
import jax.numpy as jnp
from jax import lax
from jax.experimental import pallas as pl
from jax.experimental.pallas import tpu as pltpu
from jax.experimental.pallas import tpu_sc as plsc

B = 16384
D = 128
L = 16
NC = 2
NS = 16
NW = NC * NS
BPW = B // NW
CHUNK = 128
NCHUNK = BPW // CHUNK

_mesh = plsc.VectorSubcoreMesh(core_axis_name="c", subcore_axis_name="s")


@functools.partial(
    pl.kernel,
    mesh=_mesh,
    out_type=jax.ShapeDtypeStruct((B,), jnp.float32),
    compiler_params=pltpu.CompilerParams(needs_layout_passes=False),
    scratch_types=[
        pltpu.VMEM((NCHUNK, CHUNK), jnp.int32),
        pltpu.VMEM((BPW, D), jnp.float32),
        pltpu.VMEM((D,), jnp.float32),
        pltpu.VMEM((BPW,), jnp.float32),
        pltpu.SemaphoreType.DMA,
    ],
)
def _sc_dot_gather(idx_hbm, table_hbm, user_hbm, out_hbm,
                   idx_v, rows_v, user_v, out_v, sem):
    wid = lax.axis_index("s") * NC + lax.axis_index("c")
    base = wid * BPW

    pltpu.sync_copy(idx_hbm.at[wid], idx_v)
    pltpu.sync_copy(user_hbm, user_v)

    copies = [
        pltpu.async_copy(
            table_hbm.at[idx_v.at[j]],
            rows_v.at[pl.ds(j * CHUNK, CHUNK)],
            sem,
        )
        for j in range(NCHUNK)
    ]
    for cp in copies:
        cp.wait()

    uvecs = [user_v[pl.ds(c * L, L)] for c in range(D // L)]
    lane = lax.iota(jnp.int32, L)

    def block_body(b, carry):
        vec = jnp.zeros((L,), jnp.float32)
        for i in range(L):
            row = b * L + i
            acc = uvecs[0] * rows_v[row, pl.ds(0, L)]
            for c in range(1, D // L):
                acc = acc + uvecs[c] * rows_v[row, pl.ds(c * L, L)]
            dot = jnp.sum(acc)
            vec = jnp.where(lane == i, dot, vec)
        out_v[pl.ds(b * L, L)] = vec
        return carry

    lax.fori_loop(0, BPW // L, block_body, 0)

    pltpu.sync_copy(out_v, out_hbm.at[pl.ds(base, BPW)])


def kernel(user_input, item_input, item_embedding, user_embedding):
    idx = item_input.astype(jnp.int32).reshape(NW, NCHUNK, CHUNK)
    user = user_embedding.reshape(D)
    return _sc_dot_gather(idx, item_embedding, user)
